# Initial kernel scaffold; baseline (speedup 1.0000x reference)
import jax
import jax.numpy as jnp
from jax.experimental import pallas as pl


def kernel(x, edge_index, W1, b1, W2, b2):
    raise NotImplementedError("write your pallas kernel here")


# speedup vs baseline: 12.0011x; 12.0011x over previous
import jax
import jax.numpy as jnp
from jax import lax
from jax.experimental import pallas as pl
from jax.experimental.pallas import tpu as pltpu
from jax.experimental.pallas import tpu_sc as plsc

N = 10000
E = 320000
D = 128

NC = 2
NS = 16
NW = NC * NS
EPW = E // NW
CH = 80
NCH = EPW // CH
RD = 624
TAIL = N - NS * RD
HRD = 312

_mesh = plsc.VectorSubcoreMesh(
    core_axis_name="c", subcore_axis_name="s", num_cores=NC, num_subcores=NS
)

_DEG_SCRATCH = [
    pltpu.VMEM((CH,), jnp.int32),
    pltpu.VMEM((CH, D), jnp.float32),
    pltpu.VMEM((HRD, D), jnp.float32),
    pltpu.VMEM_SHARED((N, D), jnp.float32),
]


def _deg_body(dst_hbm, out0_hbm, out1_hbm, didx, ones_v, zbuf, acc):
    cid = lax.axis_index("c")
    sid = lax.axis_index("s")
    wid = cid * NS + sid

    DV = D // 16

    def fill(i, carry):
        ones_v[i // DV, pl.ds((i % DV) * 16, 16)] = jnp.ones((16,), jnp.float32)
        return carry

    lax.fori_loop(0, CH * DV, fill, 0)

    def zfill(i, carry):
        zbuf[i // DV, pl.ds((i % DV) * 16, 16)] = jnp.zeros((16,), jnp.float32)
        return carry

    lax.fori_loop(0, HRD * DV, zfill, 0)

    for i in range(NS):
        @pl.when(sid == i)
        def _():
            pltpu.sync_copy(zbuf, acc.at[pl.ds(i * RD, HRD)])
            pltpu.sync_copy(zbuf, acc.at[pl.ds(i * RD + HRD, HRD)])

    @pl.when(sid == 0)
    def _():
        pltpu.sync_copy(zbuf.at[pl.ds(0, TAIL)], acc.at[pl.ds(NS * RD, TAIL)])

    plsc.subcore_barrier()

    def ebody(j, carry):
        off = pl.multiple_of(wid * EPW + j * CH, 8)
        pltpu.sync_copy(dst_hbm.at[pl.ds(off, CH)], didx)
        pltpu.sync_copy(ones_v, acc.at[didx], add=True)
        return carry

    lax.fori_loop(0, NCH, ebody, 0)
    plsc.subcore_barrier()

    for c, out in ((0, out0_hbm), (1, out1_hbm)):
        for i in range(NS):
            @pl.when(jnp.logical_and(cid == c, sid == i))
            def _():
                for z in range(RD // HRD):
                    pltpu.sync_copy(acc.at[pl.ds(i * RD + z * HRD, HRD)], zbuf)
                    pltpu.sync_copy(zbuf, out.at[pl.ds(i * RD + z * HRD, HRD)])

        @pl.when(jnp.logical_and(cid == c, sid == 0))
        def _():
            pltpu.sync_copy(acc.at[pl.ds(NS * RD, TAIL)], zbuf.at[pl.ds(0, TAIL)])
            pltpu.sync_copy(zbuf.at[pl.ds(0, TAIL)], out.at[pl.ds(NS * RD, TAIL)])


_deg_kernel = pl.kernel(
    _deg_body,
    out_type=[jax.ShapeDtypeStruct((N, D), jnp.float32),
              jax.ShapeDtypeStruct((N, D), jnp.float32)],
    mesh=_mesh,
    scratch_types=_DEG_SCRATCH,
)

_EDGE_SCRATCH = [
    pltpu.VMEM((CH,), jnp.int32),
    pltpu.VMEM((CH,), jnp.int32),
    pltpu.VMEM((CH, D), jnp.float32),
    pltpu.VMEM((HRD, D), jnp.float32),
    pltpu.VMEM_SHARED((N, D), jnp.float32),
    pltpu.SemaphoreType.DMA,
]


def _edge_body(p_hbm, src_hbm, dst_hbm, out0_hbm, out1_hbm,
               sidx, didx, rows, zbuf, acc, sem):
    cid = lax.axis_index("c")
    sid = lax.axis_index("s")
    wid = cid * NS + sid

    DV = D // 16

    def zfill(i, carry):
        zbuf[i // DV, pl.ds((i % DV) * 16, 16)] = jnp.zeros((16,), jnp.float32)
        return carry

    lax.fori_loop(0, HRD * DV, zfill, 0)

    for i in range(NS):
        @pl.when(sid == i)
        def _():
            pltpu.sync_copy(zbuf, acc.at[pl.ds(i * RD, HRD)])
            pltpu.sync_copy(zbuf, acc.at[pl.ds(i * RD + HRD, HRD)])

    @pl.when(sid == 0)
    def _():
        pltpu.sync_copy(zbuf.at[pl.ds(0, TAIL)], acc.at[pl.ds(NS * RD, TAIL)])

    plsc.subcore_barrier()

    def ebody(j, carry):
        off = pl.multiple_of(wid * EPW + j * CH, 8)
        pltpu.sync_copy(src_hbm.at[pl.ds(off, CH)], sidx)
        pltpu.sync_copy(dst_hbm.at[pl.ds(off, CH)], didx)
        pltpu.async_copy(p_hbm.at[sidx], rows, sem).wait()
        pltpu.sync_copy(rows, acc.at[didx], add=True)
        return carry

    lax.fori_loop(0, NCH, ebody, 0)
    plsc.subcore_barrier()

    for c, out in ((0, out0_hbm), (1, out1_hbm)):
        for i in range(NS):
            @pl.when(jnp.logical_and(cid == c, sid == i))
            def _():
                for z in range(RD // HRD):
                    pltpu.sync_copy(acc.at[pl.ds(i * RD + z * HRD, HRD)], zbuf)
                    pltpu.sync_copy(zbuf, out.at[pl.ds(i * RD + z * HRD, HRD)])

        @pl.when(jnp.logical_and(cid == c, sid == 0))
        def _():
            pltpu.sync_copy(acc.at[pl.ds(NS * RD, TAIL)], zbuf.at[pl.ds(0, TAIL)])
            pltpu.sync_copy(zbuf.at[pl.ds(0, TAIL)], out.at[pl.ds(NS * RD, TAIL)])


_edge_kernel = pl.kernel(
    _edge_body,
    out_type=[jax.ShapeDtypeStruct((N, D), jnp.float32),
              jax.ShapeDtypeStruct((N, D), jnp.float32)],
    mesh=_mesh,
    scratch_types=_EDGE_SCRATCH,
)

RB = 2000
GRID = N // RB


SQRT_D = float(D) ** 0.5


def _prep_body(d0_ref, d1_ref, x_ref, w1_ref, g_ref, p1_ref):
    dsum = jnp.sum(d0_ref[...] + d1_ref[...], axis=1, keepdims=True)
    g = SQRT_D * lax.rsqrt(dsum + float(D))
    h = jnp.dot(
        x_ref[...], w1_ref[...], preferred_element_type=jnp.float32,
        precision=lax.Precision.HIGHEST,
    )
    g_ref[...] = g
    p1_ref[...] = h * g


_prep = pl.pallas_call(
    _prep_body,
    grid=(GRID,),
    in_specs=[
        pl.BlockSpec((RB, D), lambda i: (i, 0)),
        pl.BlockSpec((RB, D), lambda i: (i, 0)),
        pl.BlockSpec((RB, D), lambda i: (i, 0)),
        pl.BlockSpec((D, D), lambda i: (0, 0)),
    ],
    out_specs=[
        pl.BlockSpec((RB, 1), lambda i: (i, 0)),
        pl.BlockSpec((RB, D), lambda i: (i, 0)),
    ],
    out_shape=[
        jax.ShapeDtypeStruct((N, 1), jnp.float32),
        jax.ShapeDtypeStruct((N, D), jnp.float32),
    ],
)


def _mid_body(s0_ref, s1_ref, p_ref, g_ref, w2_ref, b1_ref, p2_ref):
    t = (s0_ref[...] + s1_ref[...] + p_ref[...]) * g_ref[...] + b1_ref[...]
    h = jnp.maximum(t, 0.0)
    p2_ref[...] = (
        jnp.dot(
            h, w2_ref[...], preferred_element_type=jnp.float32,
            precision=lax.Precision.HIGHEST,
        )
        * g_ref[...]
    )


_mid = pl.pallas_call(
    _mid_body,
    grid=(GRID,),
    in_specs=[
        pl.BlockSpec((RB, D), lambda i: (i, 0)),
        pl.BlockSpec((RB, D), lambda i: (i, 0)),
        pl.BlockSpec((RB, D), lambda i: (i, 0)),
        pl.BlockSpec((RB, 1), lambda i: (i, 0)),
        pl.BlockSpec((D, D), lambda i: (0, 0)),
        pl.BlockSpec((1, D), lambda i: (0, 0)),
    ],
    out_specs=pl.BlockSpec((RB, D), lambda i: (i, 0)),
    out_shape=jax.ShapeDtypeStruct((N, D), jnp.float32),
)


def _fin_body(s0_ref, s1_ref, p_ref, g_ref, b2_ref, o_ref):
    o_ref[...] = (s0_ref[...] + s1_ref[...] + p_ref[...]) * g_ref[...] + b2_ref[...]


_fin = pl.pallas_call(
    _fin_body,
    grid=(GRID,),
    in_specs=[
        pl.BlockSpec((RB, D), lambda i: (i, 0)),
        pl.BlockSpec((RB, D), lambda i: (i, 0)),
        pl.BlockSpec((RB, D), lambda i: (i, 0)),
        pl.BlockSpec((RB, 1), lambda i: (i, 0)),
        pl.BlockSpec((1, D), lambda i: (0, 0)),
    ],
    out_specs=pl.BlockSpec((RB, D), lambda i: (i, 0)),
    out_shape=jax.ShapeDtypeStruct((N, D), jnp.float32),
)


def kernel(x, edge_index, W1, b1, W2, b2):
    src = edge_index[0]
    dst = edge_index[1]
    d0, d1 = _deg_kernel(dst)
    g, p1 = _prep(d0, d1, x, W1)
    s1a, s1b = _edge_kernel(p1, src, dst)
    p2 = _mid(s1a, s1b, p1, g, W2, b1.reshape(1, D))
    s2a, s2b = _edge_kernel(p2, src, dst)
    return _fin(s2a, s2b, p2, g, b2.reshape(1, D))
